# baseline (device time: 94312 ns/iter reference)
import jax
import jax.numpy as jnp
from jax import lax
from jax.experimental import pallas as pl
from jax.experimental.pallas import tpu as pltpu

N_Y = 4
V_LOC = 8192
T = 1024
D = 1024
CHUNK = T // N_Y
N_STEPS = 2 * (N_Y - 1)


def kernel(ids, E):
    y = lax.axis_index("y")
    local = ids - y * V_LOC
    oob = (local < 0) | (local >= V_LOC)
    safe = jnp.where(oob, 0, local)
    part = jnp.take(E, safe, axis=0)
    part = jnp.where(oob[:, None], 0.0, part).astype(jnp.bfloat16)
    return _ring_allreduce_y(part)


def _ring_allreduce_y(part):
    def body(part_ref, out_ref, acc_ref, comm_ref, send_sems, recv_sems):
        my_x = lax.axis_index("x")
        my_y = lax.axis_index("y")
        my_z = lax.axis_index("z")
        right = (my_y + 1) % N_Y
        left = (my_y + N_Y - 1) % N_Y

        barrier_sem = pltpu.get_barrier_semaphore()
        for nbr in (left, right):
            pl.semaphore_signal(
                barrier_sem, inc=1,
                device_id=(my_x, nbr, my_z),
                device_id_type=pl.DeviceIdType.MESH,
            )
        pl.semaphore_wait(barrier_sem, 2)

        acc_ref[...] = part_ref[...]

        for s in range(N_Y - 1):
            send_c = ((my_y + N_Y - s) % N_Y) * CHUNK
            recv_c = ((my_y + N_Y - s - 1) % N_Y) * CHUNK
            rdma = pltpu.make_async_remote_copy(
                src_ref=acc_ref.at[pl.ds(send_c, CHUNK), :],
                dst_ref=comm_ref.at[s],
                send_sem=send_sems.at[s],
                recv_sem=recv_sems.at[s],
                device_id=(my_x, right, my_z),
                device_id_type=pl.DeviceIdType.MESH,
            )
            rdma.start()
            rdma.wait()
            acc_ref[pl.ds(recv_c, CHUNK), :] += comm_ref[s]

        for s in range(N_Y - 1):
            send_c = ((my_y + 1 + N_Y - s) % N_Y) * CHUNK
            recv_c = ((my_y + N_Y - s) % N_Y) * CHUNK
            slot = (N_Y - 1) + s
            rdma = pltpu.make_async_remote_copy(
                src_ref=acc_ref.at[pl.ds(send_c, CHUNK), :],
                dst_ref=comm_ref.at[slot],
                send_sem=send_sems.at[slot],
                recv_sem=recv_sems.at[slot],
                device_id=(my_x, right, my_z),
                device_id_type=pl.DeviceIdType.MESH,
            )
            rdma.start()
            rdma.wait()
            acc_ref[pl.ds(recv_c, CHUNK), :] = comm_ref[slot]

        out_ref[...] = acc_ref[...].astype(jnp.float32)

    return pl.pallas_call(
        body,
        out_shape=jax.ShapeDtypeStruct((T, D), jnp.float32),
        in_specs=[pl.BlockSpec(memory_space=pltpu.VMEM)],
        out_specs=pl.BlockSpec(memory_space=pltpu.VMEM),
        scratch_shapes=[
            pltpu.VMEM((T, D), jnp.bfloat16),
            pltpu.VMEM((N_STEPS, CHUNK, D), jnp.bfloat16),
            pltpu.SemaphoreType.DMA((N_STEPS,)),
            pltpu.SemaphoreType.DMA((N_STEPS,)),
        ],
        compiler_params=pltpu.CompilerParams(collective_id=0),
    )(part)
